# baseline (speedup 1.0000x reference)
import dataclasses
import functools

import jax
import jax.numpy as jnp
from jax import lax
from jax.experimental import pallas as pl
from jax.experimental.pallas import tpu as pltpu
from jax.experimental.pallas import tpu_sc as plsc

_N = 1048576
_K_SC = 1048576
_POS_LB = 0.6
_NEG_UB = 0.45

_LOG_C = (
    1.000000697638299,
    -0.5000073579371714,
    0.3331793082944872,
    -0.2492948416299963,
    0.20455460255136912,
    -0.18456089482990853,
    0.11784613899531443,
)
_LN2 = 0.6931471805599453
_SQRT2 = 1.4142135623730951
_MIN_NORM = 1.1754943508222875e-38

_SC_BLK = 4096
_SC_NW = 32
_SC_L = 16


def _sc_elem(p, iou):
    pos = iou >= _POS_LB
    sel = jnp.logical_or(pos, iou <= _NEG_UB)
    arg = jnp.where(pos, p, 1.0 - p)
    ibits = plsc.bitcast(arg, jnp.int32)
    e = (ibits >> 23) - 127
    m = plsc.bitcast((ibits & 0x7FFFFF) | 0x3F800000, jnp.float32)
    big = m >= _SQRT2
    m = jnp.where(big, 0.5 * m, m)
    ef = e.astype(jnp.float32) + jnp.where(big, 1.0, 0.0)
    u = m - 1.0
    pu = jnp.full((_SC_L,), _LOG_C[6], jnp.float32)
    for c in _LOG_C[5::-1]:
        pu = pu * u + c
    lg = ef * _LN2 + u * pu
    lg = jnp.maximum(lg, -100.0)
    lg = jnp.where(arg < _MIN_NORM, -100.0, lg)
    contrib = jnp.where(sel, -lg, 0.0)
    ones = jnp.where(sel, 1.0, 0.0)
    return contrib, ones


def _sc_block(p_vmem, i_vmem, acc_s, acc_c):
    def step(k, carry):
        s, c = carry
        p = p_vmem[pl.ds(k * _SC_L, _SC_L)]
        io = i_vmem[pl.ds(k * _SC_L, _SC_L)]
        contrib, ones = _sc_elem(p, io)
        return s + contrib, c + ones

    z = jnp.zeros((_SC_L,), jnp.float32)
    s, c = lax.fori_loop(0, _SC_BLK // _SC_L, step, (z, z))
    acc_s[...] += s
    acc_c[...] += c


def _sc_partial(p_flat, i_flat):
    n = p_flat.shape[0]
    nblk = n // _SC_BLK
    mesh = plsc.VectorSubcoreMesh(core_axis_name="c", subcore_axis_name="s")
    cp = pltpu.CompilerParams()
    if "needs_layout_passes" in pltpu.CompilerParams.__dataclass_fields__:
        cp = dataclasses.replace(cp, needs_layout_passes=False)

    @functools.partial(
        pl.kernel,
        mesh=mesh,
        compiler_params=cp,
        out_type=[
            jax.ShapeDtypeStruct((_SC_NW, _SC_L), jnp.float32),
            jax.ShapeDtypeStruct((_SC_NW, _SC_L), jnp.float32),
        ],
        scratch_types=[
            pltpu.VMEM((_SC_L,), jnp.float32),
            pltpu.VMEM((_SC_L,), jnp.float32),
        ],
    )
    def sck(p_hbm, i_hbm, s_hbm, c_hbm, acc_s, acc_c):
        acc_s[...] = jnp.zeros((_SC_L,), jnp.float32)
        acc_c[...] = jnp.zeros((_SC_L,), jnp.float32)
        pltpu.emit_pipeline(
            lambda pv, iv: _sc_block(pv, iv, acc_s, acc_c),
            grid=(nblk,),
            in_specs=[
                pl.BlockSpec((_SC_BLK,), lambda i: (i,)),
                pl.BlockSpec((_SC_BLK,), lambda i: (i,)),
            ],
            out_specs=[],
            core_axis_name=("c", "s"),
            dimension_semantics=(pltpu.PARALLEL,),
        )(p_hbm, i_hbm)
        wid = lax.axis_index("s") * 2 + lax.axis_index("c")
        pltpu.sync_copy(acc_s, s_hbm.at[wid])
        pltpu.sync_copy(acc_c, c_hbm.at[wid])

    return sck(p_flat, i_flat)


def _tc_body(p_ref, i_ref, s_ref, c_ref, acc_ref):
    step = pl.program_id(0)
    p = p_ref[...]
    iou = i_ref[...]
    pos = iou >= _POS_LB
    sel = pos | (iou <= _NEG_UB)
    arg = jnp.where(pos, p, 1.0 - p)
    l = jnp.maximum(jnp.log(arg), -100.0)
    s = jnp.sum(jnp.where(sel, -l, 0.0))
    c = jnp.sum(jnp.where(sel, 1.0, 0.0))

    @pl.when(step == 0)
    def _():
        acc_ref[0] = 0.0
        acc_ref[1] = 0.0

    acc_ref[0] += s
    acc_ref[1] += c

    @pl.when(step == pl.num_programs(0) - 1)
    def _():
        s_ref[0, 0] = acc_ref[0]
        c_ref[0, 0] = acc_ref[1]


def _tc_partial(p_flat, i_flat):
    n = p_flat.shape[0]
    cols = 128
    rows = n // cols
    blk_rows = min(rows, 512)
    p2 = p_flat.reshape(rows, cols)
    i2 = i_flat.reshape(rows, cols)
    s, c = pl.pallas_call(
        _tc_body,
        grid=(rows // blk_rows,),
        in_specs=[
            pl.BlockSpec((blk_rows, cols), lambda i: (i, 0)),
            pl.BlockSpec((blk_rows, cols), lambda i: (i, 0)),
        ],
        out_specs=[
            pl.BlockSpec((1, 1), lambda i: (0, 0), memory_space=pltpu.SMEM),
            pl.BlockSpec((1, 1), lambda i: (0, 0), memory_space=pltpu.SMEM),
        ],
        out_shape=[
            jax.ShapeDtypeStruct((1, 1), jnp.float32),
            jax.ShapeDtypeStruct((1, 1), jnp.float32),
        ],
        scratch_shapes=[pltpu.SMEM((2,), jnp.float32)],
    )(p2, i2)
    return s[0, 0], c[0, 0]


@jax.jit
def kernel(pred, iou):
    p = pred.reshape(_N)
    total = jnp.float32(0.0)
    cnt = jnp.float32(0.0)
    if _K_SC > 0:
        sc_s, sc_c = _sc_partial(p[:_K_SC], iou[:_K_SC])
        total = total + jnp.sum(sc_s)
        cnt = cnt + jnp.sum(sc_c)
    if _K_SC < _N:
        tc_s, tc_c = _tc_partial(p[_K_SC:], iou[_K_SC:])
        total = total + tc_s
        cnt = cnt + tc_c
    return jnp.where(cnt > 0.0, total / cnt, jnp.float32(0.0))


# speedup vs baseline: 2.4290x; 2.4290x over previous
import dataclasses
import functools

import jax
import jax.numpy as jnp
from jax import lax
from jax.experimental import pallas as pl
from jax.experimental.pallas import tpu as pltpu
from jax.experimental.pallas import tpu_sc as plsc

_N = 1048576
_K_SC = 0
_POS_LB = 0.6
_NEG_UB = 0.45

_LOG_C = (
    1.000000697638299,
    -0.5000073579371714,
    0.3331793082944872,
    -0.2492948416299963,
    0.20455460255136912,
    -0.18456089482990853,
    0.11784613899531443,
)
_LN2 = 0.6931471805599453
_SQRT2 = 1.4142135623730951
_MIN_NORM = 1.1754943508222875e-38

_SC_BLK = 4096
_SC_NW = 32
_SC_L = 16


def _masked_bce(p, iou):
    pos = iou >= _POS_LB
    sel = jnp.logical_or(pos, iou <= _NEG_UB)
    arg = jnp.where(pos, p, 1.0 - p)
    ibits = lax.bitcast_convert_type(arg, jnp.int32)
    e = (ibits >> 23) - 127
    m = lax.bitcast_convert_type((ibits & 0x7FFFFF) | 0x3F800000, jnp.float32)
    big = m >= _SQRT2
    m = jnp.where(big, 0.5 * m, m)
    ef = e.astype(jnp.float32) + jnp.where(big, 1.0, 0.0)
    u = m - 1.0
    pu = jnp.full(u.shape, _LOG_C[6], jnp.float32)
    for c in _LOG_C[5::-1]:
        pu = pu * u + c
    lg = ef * _LN2 + u * pu
    lg = jnp.maximum(lg, -100.0)
    lg = jnp.where(arg < _MIN_NORM, -100.0, lg)
    contrib = jnp.where(sel, -lg, 0.0)
    ones = jnp.where(sel, 1.0, 0.0)
    return contrib, ones


def _sc_block(p_vmem, i_vmem, acc_s, acc_c):
    def step(k, carry):
        s, c = carry
        p = p_vmem[pl.ds(k * _SC_L, _SC_L)]
        io = i_vmem[pl.ds(k * _SC_L, _SC_L)]
        contrib, ones = _masked_bce(p, io)
        return s + contrib, c + ones

    z = jnp.zeros((_SC_L,), jnp.float32)
    s, c = lax.fori_loop(0, _SC_BLK // _SC_L, step, (z, z))
    acc_s[...] += s
    acc_c[...] += c


def _sc_partial(p_flat, i_flat):
    n = p_flat.shape[0]
    nblk = n // _SC_BLK
    mesh = plsc.VectorSubcoreMesh(core_axis_name="c", subcore_axis_name="s")
    cp = pltpu.CompilerParams()
    if "needs_layout_passes" in pltpu.CompilerParams.__dataclass_fields__:
        cp = dataclasses.replace(cp, needs_layout_passes=False)

    @functools.partial(
        pl.kernel,
        mesh=mesh,
        compiler_params=cp,
        out_type=[
            jax.ShapeDtypeStruct((_SC_NW, _SC_L), jnp.float32),
            jax.ShapeDtypeStruct((_SC_NW, _SC_L), jnp.float32),
        ],
        scratch_types=[
            pltpu.VMEM((_SC_L,), jnp.float32),
            pltpu.VMEM((_SC_L,), jnp.float32),
        ],
    )
    def sck(p_hbm, i_hbm, s_hbm, c_hbm, acc_s, acc_c):
        acc_s[...] = jnp.zeros((_SC_L,), jnp.float32)
        acc_c[...] = jnp.zeros((_SC_L,), jnp.float32)
        pltpu.emit_pipeline(
            lambda pv, iv: _sc_block(pv, iv, acc_s, acc_c),
            grid=(nblk,),
            in_specs=[
                pl.BlockSpec((_SC_BLK,), lambda i: (i,)),
                pl.BlockSpec((_SC_BLK,), lambda i: (i,)),
            ],
            out_specs=[],
            core_axis_name=("c", "s"),
            dimension_semantics=(pltpu.PARALLEL,),
        )(p_hbm, i_hbm)
        wid = lax.axis_index("s") * 2 + lax.axis_index("c")
        pltpu.sync_copy(acc_s, s_hbm.at[wid])
        pltpu.sync_copy(acc_c, c_hbm.at[wid])

    return sck(p_flat, i_flat)


def _tc_body(p_ref, i_ref, s_ref, c_ref, acc_ref):
    step = pl.program_id(0)
    p = p_ref[...]
    iou = i_ref[...]
    contrib, ones = _masked_bce(p, iou)
    s = jnp.sum(contrib)
    c = jnp.sum(ones)

    @pl.when(step == 0)
    def _():
        acc_ref[0] = 0.0
        acc_ref[1] = 0.0

    acc_ref[0] += s
    acc_ref[1] += c

    @pl.when(step == pl.num_programs(0) - 1)
    def _():
        s_ref[0, 0] = acc_ref[0]
        c_ref[0, 0] = acc_ref[1]


def _tc_partial(p_flat, i_flat):
    n = p_flat.shape[0]
    cols = 128
    rows = n // cols
    blk_rows = min(rows, 512)
    p2 = p_flat.reshape(rows, cols)
    i2 = i_flat.reshape(rows, cols)
    s, c = pl.pallas_call(
        _tc_body,
        grid=(rows // blk_rows,),
        in_specs=[
            pl.BlockSpec((blk_rows, cols), lambda i: (i, 0)),
            pl.BlockSpec((blk_rows, cols), lambda i: (i, 0)),
        ],
        out_specs=[
            pl.BlockSpec((1, 1), lambda i: (0, 0), memory_space=pltpu.SMEM),
            pl.BlockSpec((1, 1), lambda i: (0, 0), memory_space=pltpu.SMEM),
        ],
        out_shape=[
            jax.ShapeDtypeStruct((1, 1), jnp.float32),
            jax.ShapeDtypeStruct((1, 1), jnp.float32),
        ],
        scratch_shapes=[pltpu.SMEM((2,), jnp.float32)],
    )(p2, i2)
    return s[0, 0], c[0, 0]


@jax.jit
def kernel(pred, iou):
    p = pred.reshape(_N)
    total = jnp.float32(0.0)
    cnt = jnp.float32(0.0)
    if _K_SC > 0:
        sc_s, sc_c = _sc_partial(p[:_K_SC], iou[:_K_SC])
        total = total + jnp.sum(sc_s)
        cnt = cnt + jnp.sum(sc_c)
    if _K_SC < _N:
        tc_s, tc_c = _tc_partial(p[_K_SC:], iou[_K_SC:])
        total = total + tc_s
        cnt = cnt + tc_c
    return jnp.where(cnt > 0.0, total / cnt, jnp.float32(0.0))
